# Initial kernel scaffold; baseline (speedup 1.0000x reference)
import jax
import jax.numpy as jnp
from jax.experimental import pallas as pl


def kernel(x, edge_index, w_ego, b_ego, w_cut, b_cut, w_glob, b_glob, w_fc, b_fc):
    raise NotImplementedError("write your pallas kernel here")


# speedup vs baseline: 2.6571x; 2.6571x over previous
import jax
import jax.numpy as jnp
import numpy as np
from jax.experimental import pallas as pl
from jax.experimental.pallas import tpu as pltpu

_N = 10000
_NP = 10240
_BLK = 256
_KB = 10240 // _BLK


def _ego_body(a_ik, a_k, a_i, x_full, out_ref, acc_ref):
    k = pl.program_id(1)

    @pl.when(k == 0)
    def _init():
        acc_ref[...] = jnp.zeros_like(acc_ref)

    acc_ref[...] += jnp.dot(a_ik[...], a_k[...],
                            preferred_element_type=jnp.float32)

    @pl.when(k == _KB - 1)
    def _fin():
        i = pl.program_id(0)
        rows = i * _BLK + jax.lax.broadcasted_iota(jnp.int32, (_BLK, _NP), 0)
        cols = jax.lax.broadcasted_iota(jnp.int32, (_BLK, _NP), 1)
        m = ((acc_ref[...] + a_i[...].astype(jnp.float32)
              + (rows == cols).astype(jnp.float32)) > 0).astype(jnp.float32)
        esum = jnp.dot(m, x_full[...], preferred_element_type=jnp.float32)
        ecnt = jnp.sum(m, axis=1, keepdims=True)
        out_ref[...] = esum / ecnt


def _ego_pallas(a_pad, x_pad):
    grid = (_NP // _BLK, _KB)
    return pl.pallas_call(
        _ego_body,
        grid=grid,
        in_specs=[
            pl.BlockSpec((_BLK, _BLK), lambda i, k: (i, k)),
            pl.BlockSpec((_BLK, _NP), lambda i, k: (k, 0)),
            pl.BlockSpec((_BLK, _NP), lambda i, k: (i, 0)),
            pl.BlockSpec((_NP, 128), lambda i, k: (0, 0)),
        ],
        out_specs=pl.BlockSpec((_BLK, 128), lambda i, k: (i, 0)),
        out_shape=jax.ShapeDtypeStruct((_NP, 128), jnp.float32),
        scratch_shapes=[pltpu.VMEM((_BLK, _NP), jnp.float32)],
        compiler_params=pltpu.CompilerParams(
            dimension_semantics=("parallel", "arbitrary")),
    )(a_pad, a_pad, a_pad, x_pad)


def kernel(x, edge_index, w_ego, b_ego, w_cut, b_cut, w_glob, b_glob, w_fc, b_fc):
    n = x.shape[0]
    src = edge_index[0]
    dst = edge_index[1]

    a = jnp.zeros((n, n), jnp.float32).at[dst, src].set(1.0)
    a_pad = jnp.zeros((_NP, _NP), jnp.bfloat16).at[:n, :n].set(
        a.astype(jnp.bfloat16))
    x_pad = jnp.zeros((_NP, x.shape[1]), jnp.float32).at[:n].set(x)
    ego = _ego_pallas(a_pad, x_pad)[:n]

    e = src.shape[0]
    perm = jax.random.permutation(jax.random.key(1), e)
    keep = perm[e // 2:]
    ks_ = src[keep]
    kd_ = dst[keep]
    csum = jax.ops.segment_sum(x[kd_], ks_, num_segments=n)
    ccnt = jax.ops.segment_sum(jnp.ones(kd_.shape[0], x.dtype), ks_,
                               num_segments=n)
    cut = jnp.where(ccnt[:, None] > 0, csum / jnp.maximum(ccnt, 1.0)[:, None], x)

    def mp(h, W, b):
        hl = h @ W + b
        agg = jax.ops.segment_sum(hl[src], dst, num_segments=n)
        return jax.nn.relu(agg)

    ego_enc = mp(ego, w_ego, b_ego)
    cut_enc = mp(cut, w_cut, b_cut)
    glob = x @ w_glob + b_glob
    comb = jnp.concatenate([ego_enc, cut_enc, glob], axis=-1)
    out = comb @ w_fc + b_fc
    return jax.nn.log_softmax(out, axis=1)
